# baseline (speedup 1.0000x reference)
import functools

import jax
import jax.numpy as jnp
from jax import lax
from jax.experimental import pallas as pl
from jax.experimental.pallas import tpu as pltpu
from jax.experimental.pallas import tpu_sc as plsc

WIDTH = 1024
NUM_CORES = 2
NUM_SUBCORES = 16
NW = NUM_CORES * NUM_SUBCORES
CHUNK = 16
NBUF = 6
LOOK = 3


@functools.partial(jax.jit, static_argnames=("total",))
def _gather(idx_flat, pe, total):
    b_per_w = total // NW
    n_chunks = b_per_w // CHUNK
    mesh = plsc.VectorSubcoreMesh(core_axis_name="c", subcore_axis_name="s")

    @functools.partial(
        pl.kernel,
        mesh=mesh,
        out_type=jax.ShapeDtypeStruct((total, WIDTH), jnp.float32),
        scratch_types=(
            [pltpu.VMEM((b_per_w,), jnp.int32)]
            + [pltpu.VMEM((CHUNK, WIDTH), jnp.float32)] * NBUF
            + [pltpu.SemaphoreType.DMA] * (2 * NBUF)
        ),
    )
    def k(idx_hbm, table_hbm, out_hbm, idx_v, *bufs_sems):
        bufs = bufs_sems[:NBUF]
        gsems = bufs_sems[NBUF:2 * NBUF]
        ssems = bufs_sems[2 * NBUF:]
        wid = lax.axis_index("s") * NUM_CORES + lax.axis_index("c")
        base = wid * b_per_w
        pltpu.sync_copy(idx_hbm.at[pl.ds(base, b_per_w)], idx_v)

        def gather(g, b):
            off = g * CHUNK
            return pltpu.make_async_copy(
                table_hbm.at[idx_v.at[pl.ds(off, CHUNK)]], bufs[b], gsems[b])

        def store(g, b):
            off = g * CHUNK
            return pltpu.make_async_copy(
                bufs[b], out_hbm.at[pl.ds(base + off, CHUNK)], ssems[b])

        def body(g, p, store_wait=True, issue_gather=True):
            gather(g, p).wait()
            store(g, p).start()
            if store_wait:
                store(g - LOOK, (p + LOOK) % NBUF).wait()
            if issue_gather:
                gather(g + LOOK, (p + LOOK) % NBUF).start()

        for g in range(LOOK):
            gather(g, g).start()
        for g in range(LOOK):
            body(g, g, store_wait=False)

        n_steady = n_chunks - 2 * LOOK
        n_groups = n_steady // NBUF

        def group(q, carry):
            g_base = NBUF * q + LOOK
            for j in range(NBUF):
                body(g_base + j, (LOOK + j) % NBUF)
            return carry

        lax.fori_loop(0, n_groups, group, 0)

        for g in range(NBUF * n_groups + LOOK, n_chunks - LOOK):
            body(g, g % NBUF)

        for g in range(n_chunks - LOOK, n_chunks):
            body(g, g % NBUF, issue_gather=False)
        for g in range(n_chunks - LOOK, n_chunks):
            store(g, g % NBUF).wait()

    return k(idx_flat, pe)


def kernel(pos_id, pe):
    b, s = pos_id.shape
    total = b * s
    out = _gather(pos_id.reshape(total), pe, total)
    return out.reshape(b, s, WIDTH)
